# Initial kernel scaffold; baseline (speedup 1.0000x reference)
import jax
import jax.numpy as jnp
from jax.experimental import pallas as pl


def kernel(q, k, v, W_l, b_l):
    raise NotImplementedError("write your pallas kernel here")


# speedup vs baseline: 1.2705x; 1.2705x over previous
import functools

import jax
import jax.numpy as jnp
from jax import lax
from jax.experimental import pallas as pl
from jax.experimental.pallas import tpu as pltpu

H, L, D = 12, 2048, 64
BLK = 64
NBLK = L // BLK
TOPK = 4
SCALE = 1.0 / 8.0
NEG = float("-inf")


def _stats_kernel(k_ref, v_ref, s_ref, z_ref, st_ref, zt_ref):
    kx = k_ref[0]
    vx = v_ref[0]
    ck = jax.nn.softmax(kx, axis=-1)

    st = jnp.zeros((D, D), jnp.float32)
    zt = jnp.zeros((1, D), jnp.float32)
    for j in range(NBLK):
        ckb = ck[j * BLK:(j + 1) * BLK, :]
        vb = vx[j * BLK:(j + 1) * BLK, :]
        sj = lax.dot_general(ckb, vb, (((0,), (0,)), ((), ())),
                             preferred_element_type=jnp.float32)
        zj = jnp.sum(ckb, axis=0, keepdims=True)
        s_ref[0, j] = sj
        z_ref[0, j:j + 1, :] = zj
        st = st + sj
        zt = zt + zj
    st_ref[0] = st
    zt_ref[0] = zt


def _attn_kernel(idx_ref, q_ref, k_ref, v_ref, s_ref, z_ref, st_ref, zt_ref,
                 w_ref, b_ref, o_ref):
    h = pl.program_id(0)
    i = pl.program_id(1)
    qb = q_ref[0]

    bs = [idx_ref[h, i, j] for j in range(TOPK)]

    logits = []
    for bj in bs:
        kb = k_ref[0, pl.ds(bj * BLK, BLK), :]
        logits.append(lax.dot_general(qb, kb, (((1,), (1,)), ((), ())),
                                      preferred_element_type=jnp.float32)
                      * SCALE)
    lg = jnp.concatenate(logits, axis=1)
    m = jnp.max(lg, axis=-1, keepdims=True)
    p = jnp.exp(lg - m)
    denom = jnp.sum(p, axis=-1, keepdims=True)
    acc = jnp.zeros((BLK, D), jnp.float32)
    for j, bj in enumerate(bs):
        vb = v_ref[0, pl.ds(bj * BLK, BLK), :]
        acc = acc + jnp.dot(p[:, j * BLK:(j + 1) * BLK], vb,
                            preferred_element_type=jnp.float32)
    o_s = acc / denom

    cq = jax.nn.softmax(qb, axis=-1)
    sc = st_ref[0]
    zc = zt_ref[0]
    for bj in bs:
        sc = sc - s_ref[0, pl.ds(bj, 1), :, :][0]
        zc = zc - z_ref[0, pl.ds(bj, 1), :]
    num = jnp.dot(cq, sc, preferred_element_type=jnp.float32)
    den = jnp.sum(cq * zc, axis=-1, keepdims=True)
    o_l = num / (den + 1e-6)

    out = o_s + lax.dot_general(o_l, w_ref[...], (((1,), (1,)), ((), ())),
                                preferred_element_type=jnp.float32)
    o_ref[0] = out + b_ref[...]


@jax.jit
def kernel(q, k, v, W_l, b_l):
    qh = q.reshape(H, L, D)
    kh = k.reshape(H, L, D)
    vh = v.reshape(H, L, D)

    qp = q.reshape(1, H, NBLK, BLK, D).mean(axis=3)
    kp = k.reshape(1, H, NBLK, BLK, D).mean(axis=3)
    scores = jnp.einsum('bhqd,bhkd->bhqk', qp, kp)
    _, idx = jax.lax.top_k(scores, TOPK)
    idx = idx.reshape(H, NBLK, TOPK).astype(jnp.int32)

    stats = pl.pallas_call(
        _stats_kernel,
        grid=(H,),
        in_specs=[
            pl.BlockSpec((1, L, D), lambda h: (h, 0, 0)),
            pl.BlockSpec((1, L, D), lambda h: (h, 0, 0)),
        ],
        out_specs=[
            pl.BlockSpec((1, NBLK, D, D), lambda h: (h, 0, 0, 0)),
            pl.BlockSpec((1, NBLK, D), lambda h: (h, 0, 0)),
            pl.BlockSpec((1, D, D), lambda h: (h, 0, 0)),
            pl.BlockSpec((1, 1, D), lambda h: (h, 0, 0)),
        ],
        out_shape=[
            jax.ShapeDtypeStruct((H, NBLK, D, D), jnp.float32),
            jax.ShapeDtypeStruct((H, NBLK, D), jnp.float32),
            jax.ShapeDtypeStruct((H, D, D), jnp.float32),
            jax.ShapeDtypeStruct((H, 1, D), jnp.float32),
        ],
    )(kh, vh)
    s_blk, z_blk, s_tot, z_tot = stats

    out = pl.pallas_call(
        _attn_kernel,
        grid=(H, NBLK),
        in_specs=[
            pl.BlockSpec(memory_space=pltpu.SMEM),
            pl.BlockSpec((1, BLK, D), lambda h, i: (h, i, 0)),
            pl.BlockSpec((1, L, D), lambda h, i: (h, 0, 0)),
            pl.BlockSpec((1, L, D), lambda h, i: (h, 0, 0)),
            pl.BlockSpec((1, NBLK, D, D), lambda h, i: (h, 0, 0, 0)),
            pl.BlockSpec((1, NBLK, D), lambda h, i: (h, 0, 0)),
            pl.BlockSpec((1, D, D), lambda h, i: (h, 0, 0)),
            pl.BlockSpec((1, 1, D), lambda h, i: (h, 0, 0)),
            pl.BlockSpec((D, D), lambda h, i: (0, 0)),
            pl.BlockSpec((1, D), lambda h, i: (0, 0)),
        ],
        out_specs=pl.BlockSpec((1, BLK, D), lambda h, i: (h, i, 0)),
        out_shape=jax.ShapeDtypeStruct((H, L, D), jnp.float32),
    )(idx, qh, kh, vh, s_blk, z_blk, s_tot, z_tot, W_l, b_l.reshape(1, D))

    return out.reshape(1, H, L, D)


# speedup vs baseline: 2.3765x; 1.8705x over previous
import jax
import jax.numpy as jnp
from jax import lax
from jax.experimental import pallas as pl
from jax.experimental.pallas import tpu as pltpu

H, L, D = 12, 2048, 64
BLK = 64
NBLK = L // BLK
TOPK = 4
SCALE = 1.0 / 8.0


def _fused_kernel(idx_ref, q_ref, k_ref, v_ref, w_ref, b_ref, o_ref, s_ref,
                  z_ref):
    h = pl.program_id(0)

    st = jnp.zeros((D, D), jnp.float32)
    zt = jnp.zeros((1, D), jnp.float32)
    for j in range(NBLK):
        ckb = jax.nn.softmax(k_ref[0, j * BLK:(j + 1) * BLK, :], axis=-1)
        vb = v_ref[0, j * BLK:(j + 1) * BLK, :]
        sj = lax.dot_general(ckb, vb, (((0,), (0,)), ((), ())),
                             preferred_element_type=jnp.float32)
        zj = jnp.sum(ckb, axis=0, keepdims=True)
        s_ref[j] = sj
        z_ref[j:j + 1, :] = zj
        st = st + sj
        zt = zt + zj

    w = w_ref[...]
    bias = b_ref[...]

    for i in range(NBLK):
        qb = q_ref[0, i * BLK:(i + 1) * BLK, :]
        bs = [idx_ref[h, i, j] for j in range(TOPK)]

        kcat = jnp.concatenate(
            [k_ref[0, pl.ds(bj * BLK, BLK), :] for bj in bs], axis=0)
        vcat = jnp.concatenate(
            [v_ref[0, pl.ds(bj * BLK, BLK), :] for bj in bs], axis=0)
        lg = lax.dot_general(qb, kcat, (((1,), (1,)), ((), ())),
                             preferred_element_type=jnp.float32) * SCALE
        m = jnp.max(lg, axis=-1, keepdims=True)
        p = jnp.exp(lg - m)
        denom = jnp.sum(p, axis=-1, keepdims=True)
        o_s = jnp.dot(p, vcat, preferred_element_type=jnp.float32) / denom

        cq = jax.nn.softmax(qb, axis=-1)
        sc = st
        zc = zt
        for bj in bs:
            sc = sc - s_ref[pl.ds(bj, 1), :, :][0]
            zc = zc - z_ref[pl.ds(bj, 1), :]
        num = jnp.dot(cq, sc, preferred_element_type=jnp.float32)
        den = jnp.sum(cq * zc, axis=-1, keepdims=True)
        o_l = num / (den + 1e-6)

        out = o_s + lax.dot_general(o_l, w, (((1,), (1,)), ((), ())),
                                    preferred_element_type=jnp.float32)
        o_ref[0, i * BLK:(i + 1) * BLK, :] = out + bias


@jax.jit
def kernel(q, k, v, W_l, b_l):
    qh = q.reshape(H, L, D)
    kh = k.reshape(H, L, D)
    vh = v.reshape(H, L, D)

    qp = q.reshape(1, H, NBLK, BLK, D).mean(axis=3)
    kp = k.reshape(1, H, NBLK, BLK, D).mean(axis=3)
    scores = jnp.einsum('bhqd,bhkd->bhqk', qp, kp)
    _, idx = jax.lax.top_k(scores, TOPK)
    idx = idx.reshape(H, NBLK, TOPK).astype(jnp.int32)

    out = pl.pallas_call(
        _fused_kernel,
        grid=(H,),
        in_specs=[
            pl.BlockSpec(memory_space=pltpu.SMEM),
            pl.BlockSpec((1, L, D), lambda h: (h, 0, 0)),
            pl.BlockSpec((1, L, D), lambda h: (h, 0, 0)),
            pl.BlockSpec((1, L, D), lambda h: (h, 0, 0)),
            pl.BlockSpec((D, D), lambda h: (0, 0)),
            pl.BlockSpec((1, D), lambda h: (0, 0)),
        ],
        out_specs=pl.BlockSpec((1, L, D), lambda h: (h, 0, 0)),
        out_shape=jax.ShapeDtypeStruct((H, L, D), jnp.float32),
        scratch_shapes=[pltpu.VMEM((NBLK, D, D), jnp.float32),
                        pltpu.VMEM((NBLK, D), jnp.float32)],
        compiler_params=pltpu.CompilerParams(
            dimension_semantics=("parallel",)),
    )(idx, qh, kh, vh, W_l, b_l.reshape(1, D))

    return out.reshape(1, H, L, D)


# speedup vs baseline: 2.3857x; 1.0039x over previous
import jax
import jax.numpy as jnp
from jax import lax
from jax.experimental import pallas as pl
from jax.experimental.pallas import tpu as pltpu

H, L, D = 12, 2048, 64
BLK = 64
NBLK = L // BLK
TOPK = 4
SCALE = 1.0 / 8.0


def _fused_kernel(idx_ref, q_ref, k_ref, v_ref, w_ref, b_ref, o_ref, s_ref,
                  z_ref):
    h = pl.program_id(0)

    st = jnp.zeros((D, D), jnp.float32)
    zt = jnp.zeros((1, D), jnp.float32)
    for j in range(NBLK):
        ckb = jax.nn.softmax(k_ref[0, j * BLK:(j + 1) * BLK, :], axis=-1)
        vb = v_ref[0, j * BLK:(j + 1) * BLK, :]
        sj = lax.dot_general(ckb, vb, (((0,), (0,)), ((), ())),
                             preferred_element_type=jnp.float32)
        zj = jnp.sum(ckb, axis=0, keepdims=True)
        s_ref[j] = sj
        z_ref[j:j + 1, :] = zj
        st = st + sj
        zt = zt + zj

    w = w_ref[...]
    bias = b_ref[...]

    for i in range(NBLK):
        qb = q_ref[0, i * BLK:(i + 1) * BLK, :]
        bs = [idx_ref[h, i, j] for j in range(TOPK)]

        kcat = jnp.concatenate(
            [k_ref[0, pl.ds(bj * BLK, BLK), :] for bj in bs], axis=0)
        vcat = jnp.concatenate(
            [v_ref[0, pl.ds(bj * BLK, BLK), :] for bj in bs], axis=0)
        lg = lax.dot_general(qb, kcat, (((1,), (1,)), ((), ())),
                             preferred_element_type=jnp.float32) * SCALE
        m = jnp.max(lg, axis=-1, keepdims=True)
        p = jnp.exp(lg - m)
        denom = jnp.sum(p, axis=-1, keepdims=True)
        o_s = jnp.dot(p, vcat, preferred_element_type=jnp.float32) / denom

        cq = jax.nn.softmax(qb, axis=-1)
        sc = st
        zc = zt
        for bj in bs:
            sc = sc - s_ref[pl.ds(bj, 1), :, :][0]
            zc = zc - z_ref[pl.ds(bj, 1), :]
        num = jnp.dot(cq, sc, preferred_element_type=jnp.float32)
        den = jnp.sum(cq * zc, axis=-1, keepdims=True)
        o_l = num / (den + 1e-6)

        out = o_s + lax.dot_general(o_l, w, (((1,), (1,)), ((), ())),
                                    preferred_element_type=jnp.float32)
        o_ref[0, i * BLK:(i + 1) * BLK, :] = out + bias


@jax.jit
def kernel(q, k, v, W_l, b_l):
    qh = q.reshape(H, L, D)
    kh = k.reshape(H, L, D)
    vh = v.reshape(H, L, D)

    qp = q.reshape(1, H, NBLK, BLK, D).mean(axis=3)
    kp = k.reshape(1, H, NBLK, BLK, D).mean(axis=3)
    scores = jnp.einsum('bhqd,bhkd->bhqk', qp, kp)
    s_work = scores.reshape(H * NBLK, NBLK)
    lanes = jnp.arange(NBLK, dtype=jnp.int32)[None, :]
    cols = []
    for _ in range(TOPK):
        a = jnp.argmax(s_work, axis=-1).astype(jnp.int32)
        cols.append(a)
        s_work = jnp.where(lanes == a[:, None], -jnp.inf, s_work)
    idx = jnp.stack(cols, axis=-1).reshape(H, NBLK, TOPK)

    out = pl.pallas_call(
        _fused_kernel,
        grid=(H,),
        in_specs=[
            pl.BlockSpec(memory_space=pltpu.SMEM),
            pl.BlockSpec((1, L, D), lambda h: (h, 0, 0)),
            pl.BlockSpec((1, L, D), lambda h: (h, 0, 0)),
            pl.BlockSpec((1, L, D), lambda h: (h, 0, 0)),
            pl.BlockSpec((D, D), lambda h: (0, 0)),
            pl.BlockSpec((1, D), lambda h: (0, 0)),
        ],
        out_specs=pl.BlockSpec((1, L, D), lambda h: (h, 0, 0)),
        out_shape=jax.ShapeDtypeStruct((H, L, D), jnp.float32),
        scratch_shapes=[pltpu.VMEM((NBLK, D, D), jnp.float32),
                        pltpu.VMEM((NBLK, D), jnp.float32)],
        compiler_params=pltpu.CompilerParams(
            dimension_semantics=("parallel",)),
    )(idx, qh, kh, vh, W_l, b_l.reshape(1, D))

    return out.reshape(1, H, L, D)
